# baseline (speedup 1.0000x reference)
import jax
import jax.numpy as jnp
import numpy as np
from jax.experimental import pallas as pl

_D_TIME = 6
_JW = 40
_BBLK = 16


def _row_output(xb, r, bf, vtw):
    y_raw = xb[0:1, :]
    nanmask = jnp.isnan(y_raw)
    y = jnp.where(nanmask, 0.0, y_raw)
    x6 = xb[2:8, :]
    x6 = jnp.where(jnp.isnan(x6), 0.0, x6).astype(jnp.bfloat16)

    affine = jnp.dot(r, x6, preferred_element_type=jnp.float32) + bf

    t = affine[0:32, :]
    n_f = jnp.floor(t * 0.3183098861837907 + 0.5)
    parity = (n_f.astype(jnp.int32) & 1) << 31
    rr = t - n_f * 3.140625
    rr = rr - n_f * 9.67653589793e-4
    s = rr * rr
    poly = 0.9999974966049194 + s * (-0.1666516661643982 + s * (
        0.008309493772685528 + s * -0.00018446547619532794))
    val = rr * poly
    top = jax.lax.bitcast_convert_type(
        jax.lax.bitcast_convert_type(val, jnp.int32) ^ parity, jnp.float32)

    bot = affine[32:40, :]
    i = jax.lax.broadcasted_iota(jnp.int32, (8, 1), 0)
    bot = jnp.where(i == 6, y, bot)
    bot = jnp.where(i == 7, nanmask.astype(jnp.float32), bot)
    v = jnp.concatenate([top, bot], axis=0).astype(jnp.bfloat16)

    return jax.lax.dot_general(v, vtw,
                               dimension_numbers=(((0,), (0,)), ((), ())),
                               preferred_element_type=jnp.float32)


def _embed_body(x_ref, r_ref, bf_ref, vtw_ref, pos_ref, o_ref):
    r = r_ref[...]
    bf = bf_ref[...]
    vtw = vtw_ref[...]
    pos = pos_ref[...]
    for j in range(_BBLK):
        o_ref[j] = _row_output(x_ref[j], r, bf, vtw) + pos


@jax.jit
def kernel(X, given_table, pos_table, t2v_w, t2v_b, vt_W, vt_b):
    B, _, L = X.shape
    d_model = pos_table.shape[1]
    f32 = jnp.float32

    z2 = jnp.zeros((2,), f32)
    wf = jnp.concatenate([t2v_w[:, 1:].reshape(-1), z2,
                          t2v_w[:, 0], z2]).reshape(_JW, 1)
    bf = jnp.concatenate([t2v_b[:, 1:].reshape(-1), z2,
                          t2v_b[:, 0], z2]).reshape(_JW, 1)
    rr = np.arange(30)
    perm_top = (rr // 5) * 6 + (rr % 5 + 1)
    perm_bot = np.arange(6) * 6
    vtw = jnp.concatenate([
        vt_W[perm_top],
        jnp.zeros((2, d_model), f32),
        vt_W[perm_bot],
        vt_W[36][None, :],
        (given_table[0] - given_table[1])[None, :],
    ], axis=0).astype(jnp.bfloat16)
    posb = pos_table + (vt_b + given_table[1])[None, :]

    r_np = np.zeros((_JW, _D_TIME), np.float32)
    r_np[np.arange(30), np.arange(30) // 5] = 1.0
    r_np[np.arange(32, 38), np.arange(6)] = 1.0
    r = (jnp.asarray(r_np) * wf).astype(jnp.bfloat16)

    grid = (B // _BBLK,)
    out = pl.pallas_call(
        _embed_body,
        grid=grid,
        in_specs=[
            pl.BlockSpec((_BBLK, 8, L), lambda b: (b, 0, 0)),
            pl.BlockSpec((_JW, _D_TIME), lambda b: (0, 0)),
            pl.BlockSpec((_JW, 1), lambda b: (0, 0)),
            pl.BlockSpec((_JW, d_model), lambda b: (0, 0)),
            pl.BlockSpec((L, d_model), lambda b: (0, 0)),
        ],
        out_specs=pl.BlockSpec((_BBLK, L, d_model), lambda b: (b, 0, 0)),
        out_shape=jax.ShapeDtypeStruct((B, L, d_model), jnp.float32),
    )(X, r, bf, vtw, posb)
    return out
